# baseline (speedup 1.0000x reference)
import functools

import jax
import jax.numpy as jnp
from jax import lax
from jax.experimental import pallas as pl
from jax.experimental.pallas import tpu as pltpu
from jax.experimental.pallas import tpu_sc as plsc

_B, _H, _W = 8, 512, 512
_HW = _H * _W
_NC, _NS, _L = 2, 16, 16
_CH = _HW // _NS
_GROUP = 2048
_NG = _CH // _GROUP
_T = _GROUP // _L
_BPC = _B // _NC
_RC = 12582912.0
_EPS = 1e-7


def _body(flow_ref, idep_ref, out_ref, accx, accy, accw,
          fxb, fyb, idb, zb,
          wxb0, wyb0, wwb0, ixb0, wxb1, wyb1, wwb1, ixb1,
          seml0, seml1, seml2, semz0, semz1, semz2,
          semo0, semo1, semr0, semr1):
    c = lax.axis_index("c")
    s = lax.axis_index("s")
    base = s * _CH
    iof = lax.iota(jnp.int32, _L).astype(jnp.float32)
    zeros16 = jnp.zeros((_L,), jnp.float32)
    rings = ((wxb0, wyb0, wwb0, ixb0, semr0), (wxb1, wyb1, wwb1, ixb1, semr1))
    accs = (accx, accy, accw)
    semls = (seml0, seml1, seml2)
    semzs = (semz0, semz1, semz2)
    pending = [[], []]

    def zinit(i, carry):
        zb[pl.ds(i * _L, _L)] = zeros16
        return carry

    lax.fori_loop(0, _CH // _L, zinit, 0, unroll=4)

    def fire_loads(b):
        return [
            pltpu.async_copy(flow_ref.at[b, 0, pl.ds(base, _CH)], fxb, seml0),
            pltpu.async_copy(flow_ref.at[b, 1, pl.ds(base, _CH)], fyb, seml1),
            pltpu.async_copy(idep_ref.at[b, pl.ds(base, _CH)], idb, seml2),
        ]

    def fire_zeros():
        return [pltpu.async_copy(zb, acc.at[pl.ds(base, _CH)], sem)
                for acc, sem in zip(accs, semzs)]

    ldesc = fire_loads(c * _BPC)
    zdesc = fire_zeros()

    for k in range(_BPC):
        b = c * _BPC + k
        with jax.named_scope("ph_stage"):
            for dd in ldesc + zdesc:
                dd.wait()
            plsc.subcore_barrier()

        for g in range(_NG):
            wxb, wyb, wwb, ixb, sem = rings[g % 2]
            if g >= 2:
                for dd in pending[g % 2]:
                    dd.wait()

            def titer(t, carry, g=g, wxb=wxb, wyb=wyb, wwb=wwb, ixb=ixb):
                o = t * _L
                local = g * _GROUP + o
                fx = fxb[pl.ds(local, _L)]
                fy = fyb[pl.ds(local, _L)]
                dv = idb[pl.ds(local, _L)]
                x0 = o & (_W - 1)
                cx = iof + x0.astype(jnp.float32)
                yrow = s * (_CH // _W) + g * (_GROUP // _W) + (t >> 5)
                rx = (cx - fx + _RC) - _RC
                ry = (yrow.astype(jnp.float32) - fy + _RC) - _RC
                ix = rx.astype(jnp.int32)
                iy = ry.astype(jnp.int32)
                inr = ((ix.astype(jnp.uint32) < jnp.uint32(_W))
                       & (iy.astype(jnp.uint32) < jnp.uint32(_H)))
                w = jnp.where(inr, dv, jnp.float32(0.0))
                idx = jnp.where(inr, iy * _W + ix, 0)
                wxb[pl.ds(o, _L)] = fx * w
                wyb[pl.ds(o, _L)] = fy * w
                wwb[pl.ds(o, _L)] = w
                ixb[pl.ds(o, _L)] = idx
                return carry

            with jax.named_scope("ph_compute"):
                lax.fori_loop(0, _T, titer, 0)
            pending[g % 2] = [
                pltpu.async_copy(buf, acc.at[ixb], sem, add=True)
                for buf, acc in zip((wxb, wyb, wwb), accs)]

        with jax.named_scope("ph_drain"):
            for nb in range(2):
                for dd in pending[nb]:
                    dd.wait()
                pending[nb] = []
            plsc.subcore_barrier()

        with jax.named_scope("ph_readback"):
            rdesc = [pltpu.async_copy(acc.at[pl.ds(base, _CH)], buf, sem)
                     for acc, buf, sem in zip(accs, (fxb, fyb, idb), semzs)]
            for dd in rdesc:
                dd.wait()
        if k + 1 < _BPC:
            zdesc = fire_zeros()

        def fin(i, carry):
            sl = pl.ds(i * _L, _L)
            ax = fxb[sl]
            ay = fyb[sl]
            aw = idb[sl]
            inv = jnp.float32(1.0) / (aw + jnp.float32(_EPS))
            m = ax != jnp.float32(0.0)
            fxb[sl] = jnp.where(m, ax * inv, jnp.float32(0.0))
            fyb[sl] = jnp.where(m, ay * inv, jnp.float32(0.0))
            return carry

        with jax.named_scope("ph_fin"):
            lax.fori_loop(0, _CH // _L, fin, 0)
        d1 = pltpu.async_copy(fxb, out_ref.at[b, 0, pl.ds(base, _CH)], semo0)
        d2 = pltpu.async_copy(fyb, out_ref.at[b, 1, pl.ds(base, _CH)], semo1)
        d1.wait()
        d2.wait()
        if k + 1 < _BPC:
            ldesc = fire_loads(b + 1)


def kernel(flow, inv_depth):
    flow_r = flow.reshape(_B, 2, _HW)
    idep_r = inv_depth.reshape(_B, _HW)
    mesh = plsc.VectorSubcoreMesh(core_axis_name="c", subcore_axis_name="s",
                                  num_cores=_NC, num_subcores=_NS)
    ring_buf = [pltpu.VMEM((_GROUP,), jnp.float32)] * 3 + [
        pltpu.VMEM((_GROUP,), jnp.int32)]
    kfn = pl.kernel(
        _body,
        out_type=jax.ShapeDtypeStruct((_B, 2, _HW), jnp.float32),
        mesh=mesh,
        scratch_types=[
            pltpu.VMEM_SHARED((_HW,), jnp.float32),
            pltpu.VMEM_SHARED((_HW,), jnp.float32),
            pltpu.VMEM_SHARED((_HW,), jnp.float32),
            pltpu.VMEM((_CH,), jnp.float32),
            pltpu.VMEM((_CH,), jnp.float32),
            pltpu.VMEM((_CH,), jnp.float32),
            pltpu.VMEM((_CH,), jnp.float32),
            *ring_buf, *ring_buf,
            *([pltpu.SemaphoreType.DMA] * 10),
        ],
    )
    out = kfn(flow_r, idep_r)
    return out.reshape(_B, 2, _H, _W)


# speedup vs baseline: 1.0108x; 1.0108x over previous
import functools

import jax
import jax.numpy as jnp
from jax import lax
from jax.experimental import pallas as pl
from jax.experimental.pallas import tpu as pltpu
from jax.experimental.pallas import tpu_sc as plsc

_B, _H, _W = 8, 512, 512
_HW = _H * _W
_NC, _NS, _L = 2, 16, 16
_CH = _HW // _NS
_GROUP = 1024
_NG = _CH // _GROUP
_T = _GROUP // _L
_BPC = _B // _NC
_RC = 12582912.0
_EPS = 1e-7


def _body(flow_ref, idep_ref, out_ref, accx, accy, accw,
          fxb, fyb, idb, zb,
          wxb0, wyb0, wwb0, ixb0, wxb1, wyb1, wwb1, ixb1,
          seml0, seml1, seml2, semz0, semz1, semz2,
          semo0, semo1, semr0, semr1):
    c = lax.axis_index("c")
    s = lax.axis_index("s")
    base = s * _CH
    iof = lax.iota(jnp.int32, _L).astype(jnp.float32)
    zeros16 = jnp.zeros((_L,), jnp.float32)
    rings = ((wxb0, wyb0, wwb0, ixb0, semr0), (wxb1, wyb1, wwb1, ixb1, semr1))
    accs = (accx, accy, accw)
    semls = (seml0, seml1, seml2)
    semzs = (semz0, semz1, semz2)
    pending = [[], []]

    def zinit(i, carry):
        zb[pl.ds(i * _L, _L)] = zeros16
        return carry

    lax.fori_loop(0, _CH // _L, zinit, 0, unroll=4)

    def fire_loads(b):
        return [
            pltpu.async_copy(flow_ref.at[b, 0, pl.ds(base, _CH)], fxb, seml0),
            pltpu.async_copy(flow_ref.at[b, 1, pl.ds(base, _CH)], fyb, seml1),
            pltpu.async_copy(idep_ref.at[b, pl.ds(base, _CH)], idb, seml2),
        ]

    def fire_zeros():
        return [pltpu.async_copy(zb, acc.at[pl.ds(base, _CH)], sem)
                for acc, sem in zip(accs, semzs)]

    ldesc = fire_loads(c * _BPC)
    zdesc = fire_zeros()

    for k in range(_BPC):
        b = c * _BPC + k
        with jax.named_scope("ph_stage"):
            for dd in ldesc + zdesc:
                dd.wait()
            plsc.subcore_barrier()

        for g in range(_NG):
            wxb, wyb, wwb, ixb, sem = rings[g % 2]
            if g >= 2:
                for dd in pending[g % 2]:
                    dd.wait()

            def titer(t, carry, g=g, wxb=wxb, wyb=wyb, wwb=wwb, ixb=ixb):
                o = t * _L
                local = g * _GROUP + o
                fx = fxb[pl.ds(local, _L)]
                fy = fyb[pl.ds(local, _L)]
                dv = idb[pl.ds(local, _L)]
                x0 = o & (_W - 1)
                cx = iof + x0.astype(jnp.float32)
                yrow = s * (_CH // _W) + g * (_GROUP // _W) + (t >> 5)
                rx = (cx - fx + _RC) - _RC
                ry = (yrow.astype(jnp.float32) - fy + _RC) - _RC
                ix = rx.astype(jnp.int32)
                iy = ry.astype(jnp.int32)
                inr = ((ix.astype(jnp.uint32) < jnp.uint32(_W))
                       & (iy.astype(jnp.uint32) < jnp.uint32(_H)))
                w = jnp.where(inr, dv, jnp.float32(0.0))
                idx = jnp.where(inr, iy * _W + ix, 0)
                wxb[pl.ds(o, _L)] = fx * w
                wyb[pl.ds(o, _L)] = fy * w
                wwb[pl.ds(o, _L)] = w
                ixb[pl.ds(o, _L)] = idx
                return carry

            with jax.named_scope("ph_compute"):
                lax.fori_loop(0, _T, titer, 0)
            pending[g % 2] = [
                pltpu.async_copy(buf, acc.at[ixb], sem, add=True)
                for buf, acc in zip((wxb, wyb, wwb), accs)]

        with jax.named_scope("ph_drain"):
            for nb in range(2):
                for dd in pending[nb]:
                    dd.wait()
                pending[nb] = []
            plsc.subcore_barrier()

        with jax.named_scope("ph_readback"):
            rdesc = [pltpu.async_copy(acc.at[pl.ds(base, _CH)], buf, sem)
                     for acc, buf, sem in zip(accs, (fxb, fyb, idb), semzs)]
            for dd in rdesc:
                dd.wait()
        if k + 1 < _BPC:
            zdesc = fire_zeros()

        def fin(i, carry):
            sl = pl.ds(i * _L, _L)
            ax = fxb[sl]
            ay = fyb[sl]
            aw = idb[sl]
            inv = jnp.float32(1.0) / (aw + jnp.float32(_EPS))
            m = ax != jnp.float32(0.0)
            fxb[sl] = jnp.where(m, ax * inv, jnp.float32(0.0))
            fyb[sl] = jnp.where(m, ay * inv, jnp.float32(0.0))
            return carry

        with jax.named_scope("ph_fin"):
            lax.fori_loop(0, _CH // _L, fin, 0)
        d1 = pltpu.async_copy(fxb, out_ref.at[b, 0, pl.ds(base, _CH)], semo0)
        d2 = pltpu.async_copy(fyb, out_ref.at[b, 1, pl.ds(base, _CH)], semo1)
        d1.wait()
        d2.wait()
        if k + 1 < _BPC:
            ldesc = fire_loads(b + 1)


def kernel(flow, inv_depth):
    flow_r = flow.reshape(_B, 2, _HW)
    idep_r = inv_depth.reshape(_B, _HW)
    mesh = plsc.VectorSubcoreMesh(core_axis_name="c", subcore_axis_name="s",
                                  num_cores=_NC, num_subcores=_NS)
    ring_buf = [pltpu.VMEM((_GROUP,), jnp.float32)] * 3 + [
        pltpu.VMEM((_GROUP,), jnp.int32)]
    kfn = pl.kernel(
        _body,
        out_type=jax.ShapeDtypeStruct((_B, 2, _HW), jnp.float32),
        mesh=mesh,
        scratch_types=[
            pltpu.VMEM_SHARED((_HW,), jnp.float32),
            pltpu.VMEM_SHARED((_HW,), jnp.float32),
            pltpu.VMEM_SHARED((_HW,), jnp.float32),
            pltpu.VMEM((_CH,), jnp.float32),
            pltpu.VMEM((_CH,), jnp.float32),
            pltpu.VMEM((_CH,), jnp.float32),
            pltpu.VMEM((_CH,), jnp.float32),
            *ring_buf, *ring_buf,
            *([pltpu.SemaphoreType.DMA] * 10),
        ],
    )
    out = kfn(flow_r, idep_r)
    return out.reshape(_B, 2, _H, _W)
